# Initial kernel scaffold; baseline (speedup 1.0000x reference)
import functools

import jax
import jax.numpy as jnp
from jax import lax
from jax.experimental import pallas as pl
from jax.experimental.pallas import tpu as pltpu
from jax.experimental.pallas import tpu_sc as plsc

_CH = 128
_NW = 32
_BN = 2000


def _make_deg_kernel(n, e):
    nch = e // _CH
    iters = -(-nch // _NW)
    tpw = n // 16
    mesh = plsc.VectorSubcoreMesh(core_axis_name="c", subcore_axis_name="s")

    @functools.partial(
        pl.kernel,
        out_type=jax.ShapeDtypeStruct((2 * n, 2), jnp.float32),
        mesh=mesh,
        scratch_types=[
            pltpu.VMEM((_CH,), jnp.int32),
            pltpu.VMEM((_CH, 2), jnp.float32),
            pltpu.VMEM_SHARED((n, 2), jnp.float32),
        ],
    )
    def deg_kernel(dst_hbm, ones_hbm, zeros_hbm, out_hbm, didx, ones_v, acc):
        cid = lax.axis_index("c")
        sid = lax.axis_index("s")
        wid = sid * 2 + cid
        r0 = sid * tpw
        pltpu.sync_copy(ones_hbm, ones_v)
        pltpu.sync_copy(zeros_hbm, acc.at[pl.ds(r0, tpw)])
        plsc.subcore_barrier()

        @pl.loop(0, iters)
        def _(i):
            c = wid + i * _NW

            @pl.when(c < nch)
            def _():
                pltpu.sync_copy(dst_hbm.at[pl.ds(c * _CH, _CH)], didx)
                pltpu.sync_copy(ones_v, acc.at[didx], add=True)

        plsc.subcore_barrier()
        pltpu.sync_copy(acc.at[pl.ds(r0, tpw)],
                        out_hbm.at[pl.ds(cid * n + r0, tpw)])

    return deg_kernel


def _make_seg_kernel(n, e, d):
    nch = e // _CH
    iters = -(-nch // _NW)
    tpw = n // 16
    mesh = plsc.VectorSubcoreMesh(core_axis_name="c", subcore_axis_name="s")

    @functools.partial(
        pl.kernel,
        out_type=jax.ShapeDtypeStruct((2 * n, d), jnp.float32),
        mesh=mesh,
        scratch_types=[
            pltpu.VMEM((_CH,), jnp.int32),
            pltpu.VMEM((_CH,), jnp.int32),
            pltpu.VMEM((_CH, d), jnp.float32),
            pltpu.VMEM_SHARED((n, d), jnp.float32),
            pltpu.SemaphoreType.DMA,
        ],
    )
    def seg_kernel(g_hbm, src_hbm, dst_hbm, zeros_hbm, out_hbm,
                   sidx, didx, rows, acc, sem):
        cid = lax.axis_index("c")
        sid = lax.axis_index("s")
        wid = sid * 2 + cid
        r0 = sid * tpw
        pltpu.sync_copy(zeros_hbm, acc.at[pl.ds(r0, tpw)])
        plsc.subcore_barrier()

        @pl.loop(0, iters)
        def _(i):
            c = wid + i * _NW

            @pl.when(c < nch)
            def _():
                off = c * _CH
                pltpu.sync_copy(src_hbm.at[pl.ds(off, _CH)], sidx)
                pltpu.sync_copy(dst_hbm.at[pl.ds(off, _CH)], didx)
                pltpu.async_copy(g_hbm.at[sidx], rows, sem).wait()
                pltpu.sync_copy(rows, acc.at[didx], add=True)

        plsc.subcore_barrier()
        pltpu.sync_copy(acc.at[pl.ds(r0, tpw)],
                        out_hbm.at[pl.ds(cid * n + r0, tpw)])

    return seg_kernel


def _tc1_body(x_ref, d0_ref, d1_ref, w1_ref, g1_ref, dinv_ref):
    deg = d0_ref[:, 0:1] + d1_ref[:, 0:1] + 1.0
    dinv = lax.rsqrt(deg)
    h = jnp.dot(x_ref[...], w1_ref[...], preferred_element_type=jnp.float32)
    g1_ref[...] = dinv * h
    dinv_ref[...] = dinv


def _tc2_body(p0_ref, p1_ref, g1_ref, dinv_ref, b1_ref, w2_ref, g2_ref):
    dinv = dinv_ref[...]
    z = dinv * (p0_ref[...] + p1_ref[...] + g1_ref[...]) + b1_ref[...]
    h = jnp.maximum(z, 0.0)
    h2 = jnp.dot(h, w2_ref[...], preferred_element_type=jnp.float32)
    g2_ref[...] = dinv * h2


def _tc3_body(q0_ref, q1_ref, g2_ref, dinv_ref, b2_ref, out_ref):
    z = dinv_ref[...] * (q0_ref[...] + q1_ref[...] + g2_ref[...]) + b2_ref[...]
    m = jnp.max(z, axis=1, keepdims=True)
    lse = m + jnp.log(jnp.sum(jnp.exp(z - m), axis=1, keepdims=True))
    out_ref[...] = z - lse


def _row_spec(d):
    return pl.BlockSpec((_BN, d), lambda i: (i, 0))


def _full_spec(r, c):
    return pl.BlockSpec((r, c), lambda i: (0, 0))


def kernel(x, edge_index, W1, b1, W2, b2):
    n, d_in = x.shape
    e = edge_index.shape[1]
    d_hid = W1.shape[1]
    d_out = W2.shape[1]
    src = edge_index[0]
    dst = edge_index[1]

    ones2 = jnp.concatenate(
        [jnp.ones((_CH, 1), jnp.float32), jnp.zeros((_CH, 1), jnp.float32)],
        axis=1)
    zeros2 = jnp.zeros((n // 16, 2), jnp.float32)
    zeros_h = jnp.zeros((n // 16, d_hid), jnp.float32)
    zeros_o = jnp.zeros((n // 16, d_out), jnp.float32)

    degp = _make_deg_kernel(n, e)(dst, ones2, zeros2)
    d0 = degp[:n]
    d1 = degp[n:]

    grid = n // _BN
    g1, dinv = pl.pallas_call(
        _tc1_body,
        grid=(grid,),
        in_specs=[_row_spec(d_in), _row_spec(2), _row_spec(2),
                  _full_spec(d_in, d_hid)],
        out_specs=[_row_spec(d_hid), _row_spec(1)],
        out_shape=[jax.ShapeDtypeStruct((n, d_hid), jnp.float32),
                   jax.ShapeDtypeStruct((n, 1), jnp.float32)],
    )(x, d0, d1, W1)

    segp1 = _make_seg_kernel(n, e, d_hid)(g1, src, dst, zeros_h)

    g2 = pl.pallas_call(
        _tc2_body,
        grid=(grid,),
        in_specs=[_row_spec(d_hid), _row_spec(d_hid), _row_spec(d_hid),
                  _row_spec(1), _full_spec(1, d_hid),
                  _full_spec(d_hid, d_out)],
        out_specs=_row_spec(d_out),
        out_shape=jax.ShapeDtypeStruct((n, d_out), jnp.float32),
    )(segp1[:n], segp1[n:], g1, dinv, b1.reshape(1, d_hid), W2)

    segp2 = _make_seg_kernel(n, e, d_out)(g2, src, dst, zeros_o)

    out = pl.pallas_call(
        _tc3_body,
        grid=(grid,),
        in_specs=[_row_spec(d_out), _row_spec(d_out), _row_spec(d_out),
                  _row_spec(1), _full_spec(1, d_out)],
        out_specs=_row_spec(d_out),
        out_shape=jax.ShapeDtypeStruct((n, d_out), jnp.float32),
    )(segp2[:n], segp2[n:], g2, dinv, b2.reshape(1, d_out))

    return out


# speedup vs baseline: 18.9418x; 18.9418x over previous
import functools

import jax
import jax.numpy as jnp
from jax import lax
from jax.experimental import pallas as pl
from jax.experimental.pallas import tpu as pltpu
from jax.experimental.pallas import tpu_sc as plsc

_CH = 128
_NW = 32
_BN = 2000


def _make_deg_kernel(n, e):
    d = 8
    nch = e // _CH
    iters = -(-nch // _NW)
    tpw = n // 16
    mesh = plsc.VectorSubcoreMesh(core_axis_name="c", subcore_axis_name="s")

    @functools.partial(
        pl.kernel,
        out_type=jax.ShapeDtypeStruct((32, tpw, d), jnp.float32),
        mesh=mesh,
        scratch_types=[
            pltpu.VMEM((_CH,), jnp.int32),
            pltpu.VMEM((_CH, d), jnp.float32),
            pltpu.VMEM_SHARED((n, d), jnp.float32),
        ],
        compiler_params=pltpu.CompilerParams(use_tc_tiling_on_sc=False),
    )
    def deg_kernel(dst_hbm, ones_hbm, zeros_hbm, out_hbm, didx, ones_v, acc):
        cid = lax.axis_index("c")
        sid = lax.axis_index("s")
        wid = sid * 2 + cid
        r0 = sid * tpw
        pltpu.sync_copy(ones_hbm, ones_v)
        pltpu.sync_copy(zeros_hbm, acc.at[pl.ds(r0, tpw)])
        plsc.subcore_barrier()

        @pl.loop(0, iters)
        def _(i):
            c = wid + i * _NW

            @pl.when(c < nch)
            def _():
                pltpu.sync_copy(dst_hbm.at[pl.ds(c * _CH, _CH)], didx)
                pltpu.sync_copy(ones_v, acc.at[didx], add=True)

        plsc.subcore_barrier()
        pltpu.sync_copy(acc.at[pl.ds(r0, tpw)], out_hbm.at[cid * 16 + sid])

    return deg_kernel


def _make_seg_kernel(n, e, d):
    nch = e // _CH
    iters = -(-nch // _NW)
    tpw = n // 16
    mesh = plsc.VectorSubcoreMesh(core_axis_name="c", subcore_axis_name="s")

    @functools.partial(
        pl.kernel,
        out_type=jax.ShapeDtypeStruct((32, tpw, d), jnp.float32),
        mesh=mesh,
        scratch_types=[
            pltpu.VMEM((_CH,), jnp.int32),
            pltpu.VMEM((_CH,), jnp.int32),
            pltpu.VMEM((_CH, d), jnp.float32),
            pltpu.VMEM_SHARED((n, d), jnp.float32),
            pltpu.SemaphoreType.DMA,
        ],
        compiler_params=pltpu.CompilerParams(use_tc_tiling_on_sc=False),
    )
    def seg_kernel(g_hbm, src_hbm, dst_hbm, zeros_hbm, out_hbm,
                   sidx, didx, rows, acc, sem):
        cid = lax.axis_index("c")
        sid = lax.axis_index("s")
        wid = sid * 2 + cid
        r0 = sid * tpw
        pltpu.sync_copy(zeros_hbm, acc.at[pl.ds(r0, tpw)])
        plsc.subcore_barrier()

        @pl.loop(0, iters)
        def _(i):
            c = wid + i * _NW

            @pl.when(c < nch)
            def _():
                off = c * _CH
                pltpu.sync_copy(src_hbm.at[pl.ds(off, _CH)], sidx)
                pltpu.sync_copy(dst_hbm.at[pl.ds(off, _CH)], didx)
                pltpu.async_copy(g_hbm.at[sidx], rows, sem).wait()
                pltpu.sync_copy(rows, acc.at[didx], add=True)

        plsc.subcore_barrier()
        pltpu.sync_copy(acc.at[pl.ds(r0, tpw)], out_hbm.at[cid * 16 + sid])

    return seg_kernel


def _tc1_body(x_ref, d0_ref, d1_ref, w1_ref, g1_ref, dinv_ref):
    deg = d0_ref[:, 0:1] + d1_ref[:, 0:1] + 1.0
    dinv = lax.rsqrt(deg)
    h = jnp.dot(x_ref[...], w1_ref[...], preferred_element_type=jnp.float32)
    g1_ref[...] = dinv * h
    dinv_ref[...] = dinv


def _tc2_body(p0_ref, p1_ref, g1_ref, dinv_ref, b1_ref, w2_ref, g2_ref):
    dinv = dinv_ref[...]
    z = dinv * (p0_ref[...] + p1_ref[...] + g1_ref[...]) + b1_ref[...]
    h = jnp.maximum(z, 0.0)
    w2 = w2_ref[...]
    h2 = jnp.dot(h, w2, preferred_element_type=jnp.float32)
    pad = jnp.zeros((h2.shape[0], 8 - h2.shape[1]), jnp.float32)
    g2_ref[...] = jnp.concatenate([dinv * h2, pad], axis=1)


def _tc3_body(q0_ref, q1_ref, g2_ref, dinv_ref, b2_ref, out_ref):
    d_out = out_ref.shape[1]
    q = q0_ref[:, :d_out] + q1_ref[:, :d_out] + g2_ref[:, :d_out]
    z = dinv_ref[...] * q + b2_ref[...]
    m = jnp.max(z, axis=1, keepdims=True)
    lse = m + jnp.log(jnp.sum(jnp.exp(z - m), axis=1, keepdims=True))
    out_ref[...] = z - lse


def _row_spec(d):
    return pl.BlockSpec((_BN, d), lambda i: (i, 0))


def _full_spec(r, c):
    return pl.BlockSpec((r, c), lambda i: (0, 0))


def kernel(x, edge_index, W1, b1, W2, b2):
    n, d_in = x.shape
    e = edge_index.shape[1]
    d_hid = W1.shape[1]
    d_out = W2.shape[1]
    src = edge_index[0]
    dst = edge_index[1]

    ones8 = jnp.ones((_CH, 8), jnp.float32)
    zeros8 = jnp.zeros((n // 16, 8), jnp.float32)
    zeros_h = jnp.zeros((n // 16, d_hid), jnp.float32)

    degp = _make_deg_kernel(n, e)(dst, ones8, zeros8)
    d0 = degp[:16].reshape(n, 8)
    d1 = degp[16:].reshape(n, 8)

    grid = n // _BN
    g1, dinv = pl.pallas_call(
        _tc1_body,
        grid=(grid,),
        in_specs=[_row_spec(d_in), _row_spec(8), _row_spec(8),
                  _full_spec(d_in, d_hid)],
        out_specs=[_row_spec(d_hid), _row_spec(1)],
        out_shape=[jax.ShapeDtypeStruct((n, d_hid), jnp.float32),
                   jax.ShapeDtypeStruct((n, 1), jnp.float32)],
    )(x, d0, d1, W1)

    segp1 = _make_seg_kernel(n, e, d_hid)(g1, src, dst, zeros_h)
    p0 = segp1[:16].reshape(n, d_hid)
    p1 = segp1[16:].reshape(n, d_hid)

    g2 = pl.pallas_call(
        _tc2_body,
        grid=(grid,),
        in_specs=[_row_spec(d_hid), _row_spec(d_hid), _row_spec(d_hid),
                  _row_spec(1), _full_spec(1, d_hid),
                  _full_spec(d_hid, d_out)],
        out_specs=_row_spec(8),
        out_shape=jax.ShapeDtypeStruct((n, 8), jnp.float32),
    )(p0, p1, g1, dinv, b1.reshape(1, d_hid), W2)

    segp2 = _make_seg_kernel(n, e, 8)(g2, src, dst, zeros8)
    q0 = segp2[:16].reshape(n, 8)
    q1 = segp2[16:].reshape(n, 8)

    out = pl.pallas_call(
        _tc3_body,
        grid=(grid,),
        in_specs=[_row_spec(8), _row_spec(8), _row_spec(8),
                  _row_spec(1), _full_spec(1, d_out)],
        out_specs=_row_spec(d_out),
        out_shape=jax.ShapeDtypeStruct((n, d_out), jnp.float32),
    )(q0, q1, g2, dinv, b2.reshape(1, d_out))

    return out


# speedup vs baseline: 43.0502x; 2.2728x over previous
import functools

import jax
import jax.numpy as jnp
from jax import lax
from jax.experimental import pallas as pl
from jax.experimental.pallas import tpu as pltpu
from jax.experimental.pallas import tpu_sc as plsc

_CH = 128
_MAXSUB = 8
_NW = 32
_BN = 2000


def _padded_edges(e):
    blk = _CH * _MAXSUB * _NW
    return -(-e // blk) * blk


def _make_deg_kernel(n, e):
    sub = 8
    k = _CH * sub
    ep = _padded_edges(e)
    itb = ep // (k * _NW)
    tpw = n // 16
    npad = n + 8
    mesh = plsc.VectorSubcoreMesh(core_axis_name="c", subcore_axis_name="s")

    @functools.partial(
        pl.kernel,
        out_type=jax.ShapeDtypeStruct((32, tpw, 8), jnp.float32),
        mesh=mesh,
        scratch_types=[
            pltpu.VMEM((2, sub, _CH), jnp.int32),
            pltpu.VMEM((_CH, 8), jnp.float32),
            pltpu.VMEM_SHARED((npad, 8), jnp.float32),
            pltpu.SemaphoreType.DMA,
            pltpu.SemaphoreType.DMA,
        ],
        compiler_params=pltpu.CompilerParams(use_tc_tiling_on_sc=False),
    )
    def deg_kernel(dst_hbm, ones_hbm, zeros_hbm, out_hbm,
                   didx, ones_v, acc, isem, ssem):
        cid = lax.axis_index("c")
        sid = lax.axis_index("s")
        wid = sid * 2 + cid
        r0 = sid * tpw
        pltpu.sync_copy(ones_hbm, ones_v)
        pltpu.sync_copy(zeros_hbm, acc.at[pl.ds(r0, tpw)])
        plsc.subcore_barrier()

        def fire_idx(i, slot):
            blk = wid + i * _NW
            pltpu.async_copy(
                dst_hbm.at[pl.ds(blk * sub, sub)], didx.at[slot], isem)

        def wait_idx(slot):
            pltpu.make_async_copy(
                dst_hbm.at[pl.ds(0, sub)], didx.at[slot], isem).wait()

        fire_idx(0, 0)

        @pl.loop(0, itb)
        def _(i):
            slot = lax.rem(i, 2)
            wait_idx(slot)

            @pl.when(i + 1 < itb)
            def _():
                fire_idx(i + 1, 1 - slot)

            descs = [pltpu.async_copy(ones_v, acc.at[didx.at[slot, j]],
                                      ssem, add=True) for j in range(sub)]
            for dsc in descs:
                dsc.wait()

        plsc.subcore_barrier()
        pltpu.sync_copy(acc.at[pl.ds(r0, tpw)], out_hbm.at[cid * 16 + sid])

    return deg_kernel


def _make_seg_kernel(n, e, d):
    sub = 8
    k = _CH * sub
    ep = _padded_edges(e)
    itb = ep // (k * _NW)
    tpw = n // 16
    npad = n + 8
    mesh = plsc.VectorSubcoreMesh(core_axis_name="c", subcore_axis_name="s")

    @functools.partial(
        pl.kernel,
        out_type=jax.ShapeDtypeStruct((32, tpw, d), jnp.float32),
        mesh=mesh,
        scratch_types=[
            pltpu.VMEM((2, sub, _CH), jnp.int32),
            pltpu.VMEM((2, sub, _CH), jnp.int32),
            pltpu.VMEM((k, d), jnp.float32),
            pltpu.VMEM_SHARED((npad, d), jnp.float32),
            pltpu.SemaphoreType.DMA,
            pltpu.SemaphoreType.DMA,
            pltpu.SemaphoreType.DMA,
        ],
        compiler_params=pltpu.CompilerParams(use_tc_tiling_on_sc=False),
    )
    def seg_kernel(g_hbm, src_hbm, dst_hbm, zeros_hbm, out_hbm,
                   sidx, didx, rows, acc, isem, gsem, ssem):
        cid = lax.axis_index("c")
        sid = lax.axis_index("s")
        wid = sid * 2 + cid
        r0 = sid * tpw
        pltpu.sync_copy(zeros_hbm, acc.at[pl.ds(r0, tpw)])
        plsc.subcore_barrier()

        def fire_idx(i, slot):
            blk = wid + i * _NW
            pltpu.async_copy(
                src_hbm.at[pl.ds(blk * sub, sub)], sidx.at[slot], isem)
            pltpu.async_copy(
                dst_hbm.at[pl.ds(blk * sub, sub)], didx.at[slot], isem)

        def wait_idx(slot):
            pltpu.make_async_copy(
                src_hbm.at[pl.ds(0, sub)], sidx.at[slot], isem).wait()
            pltpu.make_async_copy(
                dst_hbm.at[pl.ds(0, sub)], didx.at[slot], isem).wait()

        def fire_gathers(slot, js):
            return [pltpu.async_copy(g_hbm.at[sidx.at[slot, j]],
                                     rows.at[pl.ds(j * _CH, _CH)], gsem)
                    for j in js]

        def fire_scatters(slot, js):
            return [pltpu.async_copy(rows.at[pl.ds(j * _CH, _CH)],
                                     acc.at[didx.at[slot, j]], ssem, add=True)
                    for j in js]

        fire_idx(0, 0)

        @pl.loop(0, itb)
        def _(i):
            slot = lax.rem(i, 2)
            wait_idx(slot)

            @pl.when(i + 1 < itb)
            def _():
                fire_idx(i + 1, 1 - slot)

            half = sub // 2
            ga = fire_gathers(slot, range(half))
            for dsc in ga:
                dsc.wait()
            sa = fire_scatters(slot, range(half))
            gb = fire_gathers(slot, range(half, sub))
            for dsc in gb:
                dsc.wait()
            sb = fire_scatters(slot, range(half, sub))
            for dsc in sa + sb:
                dsc.wait()

        plsc.subcore_barrier()
        pltpu.sync_copy(acc.at[pl.ds(r0, tpw)], out_hbm.at[cid * 16 + sid])

    return seg_kernel


def _tc1_body(x_ref, d0_ref, d1_ref, w1_ref, g1_ref, dinv_ref):
    deg = d0_ref[:, 0:1] + d1_ref[:, 0:1] + 1.0
    dinv = lax.rsqrt(deg)
    h = jnp.dot(x_ref[...], w1_ref[...], preferred_element_type=jnp.float32)
    g1_ref[...] = dinv * h
    dinv_ref[...] = dinv


def _tc2_body(p0_ref, p1_ref, g1_ref, dinv_ref, b1_ref, w2_ref, g2_ref):
    dinv = dinv_ref[...]
    z = dinv * (p0_ref[...] + p1_ref[...] + g1_ref[...]) + b1_ref[...]
    h = jnp.maximum(z, 0.0)
    w2 = w2_ref[...]
    h2 = jnp.dot(h, w2, preferred_element_type=jnp.float32)
    pad = jnp.zeros((h2.shape[0], 8 - h2.shape[1]), jnp.float32)
    g2_ref[...] = jnp.concatenate([dinv * h2, pad], axis=1)


def _tc3_body(q0_ref, q1_ref, g2_ref, dinv_ref, b2_ref, out_ref):
    d_out = out_ref.shape[1]
    q = q0_ref[:, :d_out] + q1_ref[:, :d_out] + g2_ref[:, :d_out]
    z = dinv_ref[...] * q + b2_ref[...]
    m = jnp.max(z, axis=1, keepdims=True)
    lse = m + jnp.log(jnp.sum(jnp.exp(z - m), axis=1, keepdims=True))
    out_ref[...] = z - lse


def _row_spec(d):
    return pl.BlockSpec((_BN, d), lambda i: (i, 0))


def _full_spec(r, c):
    return pl.BlockSpec((r, c), lambda i: (0, 0))


def _pad_rows(a):
    return jnp.concatenate(
        [a, jnp.zeros((8, a.shape[1]), jnp.float32)], axis=0)


def kernel(x, edge_index, W1, b1, W2, b2):
    n, d_in = x.shape
    e = edge_index.shape[1]
    d_hid = W1.shape[1]
    d_out = W2.shape[1]
    ep = _padded_edges(e)
    pad_idx = jnp.full((ep - e,), n, jnp.int32)
    src = jnp.concatenate([edge_index[0], pad_idx]).reshape(ep // _CH, _CH)
    dst = jnp.concatenate([edge_index[1], pad_idx]).reshape(ep // _CH, _CH)

    ones8 = jnp.ones((_CH, 8), jnp.float32)
    zeros8 = jnp.zeros((n // 16, 8), jnp.float32)
    zeros_h = jnp.zeros((n // 16, d_hid), jnp.float32)

    degp = _make_deg_kernel(n, e)(dst, ones8, zeros8)
    d0 = degp[:16].reshape(n, 8)
    d1 = degp[16:].reshape(n, 8)

    grid = n // _BN
    g1, dinv = pl.pallas_call(
        _tc1_body,
        grid=(grid,),
        in_specs=[_row_spec(d_in), _row_spec(8), _row_spec(8),
                  _full_spec(d_in, d_hid)],
        out_specs=[_row_spec(d_hid), _row_spec(1)],
        out_shape=[jax.ShapeDtypeStruct((n, d_hid), jnp.float32),
                   jax.ShapeDtypeStruct((n, 1), jnp.float32)],
    )(x, d0, d1, W1)

    segp1 = _make_seg_kernel(n, e, d_hid)(_pad_rows(g1), src, dst, zeros_h)
    p0 = segp1[:16].reshape(n, d_hid)
    p1 = segp1[16:].reshape(n, d_hid)

    g2 = pl.pallas_call(
        _tc2_body,
        grid=(grid,),
        in_specs=[_row_spec(d_hid), _row_spec(d_hid), _row_spec(d_hid),
                  _row_spec(1), _full_spec(1, d_hid),
                  _full_spec(d_hid, d_out)],
        out_specs=_row_spec(8),
        out_shape=jax.ShapeDtypeStruct((n, 8), jnp.float32),
    )(p0, p1, g1, dinv, b1.reshape(1, d_hid), W2)

    segp2 = _make_seg_kernel(n, e, 8)(_pad_rows(g2), src, dst, zeros8)
    q0 = segp2[:16].reshape(n, 8)
    q1 = segp2[16:].reshape(n, 8)

    out = pl.pallas_call(
        _tc3_body,
        grid=(grid,),
        in_specs=[_row_spec(8), _row_spec(8), _row_spec(8),
                  _row_spec(1), _full_spec(1, d_out)],
        out_specs=_row_spec(d_out),
        out_shape=jax.ShapeDtypeStruct((n, d_out), jnp.float32),
    )(q0, q1, g2, dinv, b2.reshape(1, d_out))

    return out


# speedup vs baseline: 53.3405x; 1.2390x over previous
import functools

import jax
import jax.numpy as jnp
from jax import lax
from jax.experimental import pallas as pl
from jax.experimental.pallas import tpu as pltpu
from jax.experimental.pallas import tpu_sc as plsc

_CH = 128
_SUB = 8
_NW = 32
_BN = 2000
_D = 16


def _padded_edges(e):
    blk = _CH * _SUB * _NW
    return -(-e // blk) * blk


def _make_deg_kernel(n, e):
    k = _CH * _SUB
    ep = _padded_edges(e)
    itb = ep // (k * _NW)
    tpw = n // 16
    npad = n + 8
    mesh = plsc.VectorSubcoreMesh(core_axis_name="c", subcore_axis_name="s")

    @functools.partial(
        pl.kernel,
        out_type=jax.ShapeDtypeStruct((32, tpw, _D), jnp.float32),
        mesh=mesh,
        scratch_types=[
            pltpu.VMEM((2, _SUB, _CH), jnp.int32),
            pltpu.VMEM((_CH, _D), jnp.float32),
            pltpu.VMEM_SHARED((npad, _D), jnp.float32),
            pltpu.SemaphoreType.DMA,
            pltpu.SemaphoreType.DMA,
        ],
        compiler_params=pltpu.CompilerParams(use_tc_tiling_on_sc=False),
    )
    def deg_kernel(dst_hbm, ones_hbm, zeros_hbm, out_hbm,
                   didx, ones_v, acc, isem, ssem):
        cid = lax.axis_index("c")
        sid = lax.axis_index("s")
        wid = sid * 2 + cid
        r0 = sid * tpw
        pltpu.sync_copy(ones_hbm, ones_v)
        pltpu.sync_copy(zeros_hbm, acc.at[pl.ds(r0, tpw)])
        plsc.subcore_barrier()

        def fire_idx(i, slot):
            blk = wid + i * _NW
            pltpu.async_copy(
                dst_hbm.at[pl.ds(blk * _SUB, _SUB)], didx.at[slot], isem)

        def wait_idx(slot):
            pltpu.make_async_copy(
                dst_hbm.at[pl.ds(0, _SUB)], didx.at[slot], isem).wait()

        fire_idx(0, 0)

        @pl.loop(0, itb)
        def _(i):
            slot = lax.rem(i, 2)
            wait_idx(slot)

            @pl.when(i + 1 < itb)
            def _():
                fire_idx(i + 1, 1 - slot)

            descs = [pltpu.async_copy(ones_v, acc.at[didx.at[slot, j]],
                                      ssem, add=True) for j in range(_SUB)]
            for dsc in descs:
                dsc.wait()

        plsc.subcore_barrier()
        pltpu.sync_copy(acc.at[pl.ds(r0, tpw)], out_hbm.at[cid * 16 + sid])

    return deg_kernel


def _make_seg_kernel(n, e):
    k = _CH * _SUB
    ep = _padded_edges(e)
    itb = ep // (k * _NW)
    tpw = n // 16
    npad = n + 8
    mesh = plsc.VectorSubcoreMesh(core_axis_name="c", subcore_axis_name="s")

    @functools.partial(
        pl.kernel,
        out_type=jax.ShapeDtypeStruct((32, tpw, _D), jnp.float32),
        mesh=mesh,
        scratch_types=[
            pltpu.VMEM((2, _SUB, _CH), jnp.int32),
            pltpu.VMEM((2, _SUB, _CH), jnp.int32),
            pltpu.VMEM((k, _D), jnp.float32),
            pltpu.VMEM_SHARED((npad, _D), jnp.float32),
            pltpu.SemaphoreType.DMA,
            pltpu.SemaphoreType.DMA,
            pltpu.SemaphoreType.DMA,
        ],
        compiler_params=pltpu.CompilerParams(use_tc_tiling_on_sc=False),
    )
    def seg_kernel(g_hbm, src_hbm, dst_hbm, zeros_hbm, out_hbm,
                   sidx, didx, rows, acc, isem, gsem, ssem):
        cid = lax.axis_index("c")
        sid = lax.axis_index("s")
        wid = sid * 2 + cid
        r0 = sid * tpw
        pltpu.sync_copy(zeros_hbm, acc.at[pl.ds(r0, tpw)])
        plsc.subcore_barrier()

        def fire_idx(i, slot):
            blk = wid + i * _NW
            pltpu.async_copy(
                src_hbm.at[pl.ds(blk * _SUB, _SUB)], sidx.at[slot], isem)
            pltpu.async_copy(
                dst_hbm.at[pl.ds(blk * _SUB, _SUB)], didx.at[slot], isem)

        def wait_idx(slot):
            pltpu.make_async_copy(
                src_hbm.at[pl.ds(0, _SUB)], sidx.at[slot], isem).wait()
            pltpu.make_async_copy(
                dst_hbm.at[pl.ds(0, _SUB)], didx.at[slot], isem).wait()

        def fire_gathers(slot, js):
            return [pltpu.async_copy(g_hbm.at[sidx.at[slot, j]],
                                     rows.at[pl.ds(j * _CH, _CH)], gsem)
                    for j in js]

        def fire_scatters(slot, js):
            return [pltpu.async_copy(rows.at[pl.ds(j * _CH, _CH)],
                                     acc.at[didx.at[slot, j]], ssem, add=True)
                    for j in js]

        fire_idx(0, 0)

        @pl.loop(0, itb)
        def _(i):
            slot = lax.rem(i, 2)
            wait_idx(slot)

            @pl.when(i + 1 < itb)
            def _():
                fire_idx(i + 1, 1 - slot)

            half = _SUB // 2
            ga = fire_gathers(slot, range(half))
            for dsc in ga:
                dsc.wait()
            sa = fire_scatters(slot, range(half))
            gb = fire_gathers(slot, range(half, _SUB))
            for dsc in gb:
                dsc.wait()
            sb = fire_scatters(slot, range(half, _SUB))
            for dsc in sa + sb:
                dsc.wait()

        plsc.subcore_barrier()
        pltpu.sync_copy(acc.at[pl.ds(r0, tpw)], out_hbm.at[cid * 16 + sid])

    return seg_kernel


def _tc1_body(x8_ref, d0_ref, d1_ref, w1e_ref, g1_ref, dinv_ref):
    dinv = lax.rsqrt(d0_ref[...] + d1_ref[...] + 1.0)
    h = jnp.dot(x8_ref[...], w1e_ref[...],
                preferred_element_type=jnp.float32)
    g1_ref[...] = dinv * h
    dinv_ref[...] = dinv


def _tc2_body(p0_ref, p1_ref, g1_ref, dinv_ref, b1_ref, w2e_ref, g2_ref):
    dinv = dinv_ref[...]
    z = dinv * (p0_ref[...] + p1_ref[...] + g1_ref[...]) + b1_ref[...]
    h = jnp.maximum(z, 0.0)
    h2 = jnp.dot(h, w2e_ref[...], preferred_element_type=jnp.float32)
    g2_ref[...] = dinv * h2


def _tc3_body(q0_ref, q1_ref, g2_ref, dinv_ref, b2_ref, e0_ref, e1_ref,
              o0_ref, o1_ref):
    s = dinv_ref[...] * (q0_ref[...] + q1_ref[...] + g2_ref[...])
    z0 = jnp.dot(s, e0_ref[...], preferred_element_type=jnp.float32)
    z1 = jnp.dot(s, e1_ref[...], preferred_element_type=jnp.float32)
    z0 = z0 + b2_ref[0, 0]
    z1 = z1 + b2_ref[0, 1]
    m = jnp.maximum(z0, z1)
    lse = m + jnp.log(jnp.exp(z0 - m) + jnp.exp(z1 - m))
    o0_ref[...] = z0 - lse
    o1_ref[...] = z1 - lse


def kernel(x, edge_index, W1, b1, W2, b2):
    n, d_in = x.shape
    e = edge_index.shape[1]
    d_hid = W1.shape[1]
    d_out = W2.shape[1]
    ep = _padded_edges(e)
    src = jnp.concatenate(
        [edge_index[0], jnp.zeros((ep - e,), jnp.int32)]).reshape(-1, _CH)
    dst = jnp.concatenate(
        [edge_index[1], jnp.full((ep - e,), n, jnp.int32)]).reshape(-1, _CH)

    ones16 = jnp.ones((_CH, _D), jnp.float32)
    zeros16 = jnp.zeros((n // 16, _D), jnp.float32)
    dn = n // 8

    deg_k = _make_deg_kernel(n, e)
    seg_k = _make_seg_kernel(n, e)

    degp = deg_k(dst, ones16, zeros16)
    d0 = degp[:16].reshape(dn, 128)
    d1 = degp[16:].reshape(dn, 128)

    x8 = x.reshape(dn, 8 * d_in)
    w1e = jnp.kron(jnp.eye(8, dtype=jnp.float32), W1)

    f32 = jnp.float32
    g1d, dinvd = pl.pallas_call(
        _tc1_body,
        out_shape=[jax.ShapeDtypeStruct((dn, 128), f32),
                   jax.ShapeDtypeStruct((dn, 128), f32)],
    )(x8, d0, d1, w1e)

    segp1 = seg_k(g1d.reshape(n, _D), src, dst, zeros16)
    p0 = segp1[:16].reshape(dn, 128)
    p1 = segp1[16:].reshape(dn, 128)

    b1e = jnp.tile(b1, 128 // d_hid).reshape(1, 128)
    w2e = jnp.kron(jnp.eye(128 // _D, dtype=f32),
                   jnp.pad(W2, ((0, 0), (0, _D - d_out))))

    g2d = pl.pallas_call(
        _tc2_body,
        out_shape=jax.ShapeDtypeStruct((dn, 128), f32),
    )(p0, p1, g1d, dinvd, b1e, w2e)

    segp2 = seg_k(g2d.reshape(n, _D), src, dst, zeros16)
    q0 = segp2[:16].reshape(dn, 128)
    q1 = segp2[16:].reshape(dn, 128)

    lane = jnp.arange(128)
    node = jnp.arange(8)
    e0 = (lane[:, None] == node[None, :] * _D).astype(f32)
    e1 = (lane[:, None] == node[None, :] * _D + 1).astype(f32)

    o0, o1 = pl.pallas_call(
        _tc3_body,
        out_shape=[jax.ShapeDtypeStruct((dn, 8), f32),
                   jax.ShapeDtypeStruct((dn, 8), f32)],
    )(q0, q1, g2d, dinvd, b2.reshape(1, d_out), e0, e1)

    return jnp.concatenate(
        [o0.reshape(n, 1), o1.reshape(n, 1)], axis=1)


# speedup vs baseline: 58.8455x; 1.1032x over previous
import functools

import jax
import jax.numpy as jnp
from jax import lax
from jax.experimental import pallas as pl
from jax.experimental.pallas import tpu as pltpu
from jax.experimental.pallas import tpu_sc as plsc

_CH = 128
_SUB = 8
_NW = 32
_BN = 2000
_D = 16


def _padded_edges(e):
    blk = _CH * _SUB * _NW
    return -(-e // blk) * blk


def _make_deg_kernel(n, e):
    k = _CH * _SUB
    ep = _padded_edges(e)
    itb = ep // (k * _NW)
    tpw = n // 16
    npad = n + 8
    mesh = plsc.VectorSubcoreMesh(core_axis_name="c", subcore_axis_name="s")

    @functools.partial(
        pl.kernel,
        out_type=jax.ShapeDtypeStruct((32, tpw, _D), jnp.float32),
        mesh=mesh,
        scratch_types=[
            pltpu.VMEM((2, _SUB, _CH), jnp.int32),
            pltpu.VMEM((_CH, _D), jnp.float32),
            pltpu.VMEM_SHARED((npad, _D), jnp.float32),
            pltpu.SemaphoreType.DMA,
            pltpu.SemaphoreType.DMA,
        ],
        compiler_params=pltpu.CompilerParams(use_tc_tiling_on_sc=False),
    )
    def deg_kernel(dst_hbm, ones_hbm, zeros_hbm, out_hbm,
                   didx, ones_v, acc, isem, ssem):
        cid = lax.axis_index("c")
        sid = lax.axis_index("s")
        wid = sid * 2 + cid
        r0 = sid * tpw
        pltpu.sync_copy(ones_hbm, ones_v)
        pltpu.sync_copy(zeros_hbm, acc.at[pl.ds(r0, tpw)])
        plsc.subcore_barrier()

        def fire_idx(i, slot):
            blk = wid + i * _NW
            pltpu.async_copy(
                dst_hbm.at[pl.ds(blk * _SUB, _SUB)], didx.at[slot], isem)

        def wait_idx(slot):
            pltpu.make_async_copy(
                dst_hbm.at[pl.ds(0, _SUB)], didx.at[slot], isem).wait()

        fire_idx(0, 0)

        @pl.loop(0, itb)
        def _(i):
            slot = lax.rem(i, 2)
            wait_idx(slot)

            @pl.when(i + 1 < itb)
            def _():
                fire_idx(i + 1, 1 - slot)

            descs = [pltpu.async_copy(ones_v, acc.at[didx.at[slot, j]],
                                      ssem, add=True) for j in range(_SUB)]
            for dsc in descs:
                dsc.wait()

        plsc.subcore_barrier()
        pltpu.sync_copy(acc.at[pl.ds(r0, tpw)], out_hbm.at[cid * 16 + sid])

    return deg_kernel


def _make_seg_kernel(n, e):
    k = _CH * _SUB
    ep = _padded_edges(e)
    itb = ep // (k * _NW)
    tpw = n // 16
    npad = n + 8
    mesh = plsc.VectorSubcoreMesh(core_axis_name="c", subcore_axis_name="s")

    @functools.partial(
        pl.kernel,
        out_type=jax.ShapeDtypeStruct((32, tpw, _D), jnp.float32),
        mesh=mesh,
        scratch_types=[
            pltpu.VMEM((2, _SUB, _CH), jnp.int32),
            pltpu.VMEM((2, _SUB, _CH), jnp.int32),
            pltpu.VMEM((k, _D), jnp.float32),
            pltpu.VMEM_SHARED((npad, _D), jnp.float32),
            pltpu.SemaphoreType.DMA,
            pltpu.SemaphoreType.DMA,
            pltpu.SemaphoreType.DMA,
        ],
        compiler_params=pltpu.CompilerParams(use_tc_tiling_on_sc=False),
    )
    def seg_kernel(g_hbm, src_hbm, dst_hbm, zeros_hbm, out_hbm,
                   sidx, didx, rows, acc, isem, gsem, ssem):
        cid = lax.axis_index("c")
        sid = lax.axis_index("s")
        wid = sid * 2 + cid
        r0 = sid * tpw
        pltpu.sync_copy(zeros_hbm, acc.at[pl.ds(r0, tpw)])
        plsc.subcore_barrier()

        def fire_idx(i, slot):
            blk = wid + i * _NW
            pltpu.async_copy(
                src_hbm.at[pl.ds(blk * _SUB, _SUB)], sidx.at[slot], isem)
            pltpu.async_copy(
                dst_hbm.at[pl.ds(blk * _SUB, _SUB)], didx.at[slot], isem)

        def wait_idx(slot):
            pltpu.make_async_copy(
                src_hbm.at[pl.ds(0, _SUB)], sidx.at[slot], isem).wait()
            pltpu.make_async_copy(
                dst_hbm.at[pl.ds(0, _SUB)], didx.at[slot], isem).wait()

        def fire_gathers(slot, js):
            return [pltpu.async_copy(g_hbm.at[sidx.at[slot, j]],
                                     rows.at[pl.ds(j * _CH, _CH)], gsem)
                    for j in js]

        def fire_scatters(slot, js):
            return [pltpu.async_copy(rows.at[pl.ds(j * _CH, _CH)],
                                     acc.at[didx.at[slot, j]], ssem, add=True)
                    for j in js]

        fire_idx(0, 0)

        @pl.loop(0, itb)
        def _(i):
            slot = lax.rem(i, 2)
            wait_idx(slot)

            @pl.when(i + 1 < itb)
            def _():
                fire_idx(i + 1, 1 - slot)

            half = _SUB // 2
            ga = fire_gathers(slot, range(half))
            for dsc in ga:
                dsc.wait()
            sa = fire_scatters(slot, range(half))
            gb = fire_gathers(slot, range(half, _SUB))
            for dsc in gb:
                dsc.wait()
            sb = fire_scatters(slot, range(half, _SUB))
            for dsc in sa + sb:
                dsc.wait()

        plsc.subcore_barrier()
        pltpu.sync_copy(acc.at[pl.ds(r0, tpw)], out_hbm.at[cid * 16 + sid])

    return seg_kernel


def _tc1_body(x8_ref, d0_ref, d1_ref, w1e_ref, g1_ref, dinv_ref):
    dinv = lax.rsqrt(d0_ref[...] + d1_ref[...] + 1.0)
    h = jnp.dot(x8_ref[...], w1e_ref[...],
                preferred_element_type=jnp.float32)
    g1_ref[...] = dinv * h
    dinv_ref[...] = dinv


def _tc2_body(p0_ref, p1_ref, g1_ref, dinv_ref, b1_ref, w2e_ref, g2_ref):
    dinv = dinv_ref[...]
    z = dinv * (p0_ref[...] + p1_ref[...] + g1_ref[...]) + b1_ref[...]
    h = jnp.maximum(z, 0.0)
    h2 = jnp.dot(h, w2e_ref[...], preferred_element_type=jnp.float32)
    g2_ref[...] = dinv * h2


def _tc3_body(q0_ref, q1_ref, g2_ref, dinv_ref, b2_ref, e0_ref, e1_ref,
              o0_ref, o1_ref):
    s = dinv_ref[...] * (q0_ref[...] + q1_ref[...] + g2_ref[...])
    z0 = jnp.dot(s, e0_ref[...], preferred_element_type=jnp.float32)
    z1 = jnp.dot(s, e1_ref[...], preferred_element_type=jnp.float32)
    z0 = z0 + b2_ref[0, 0]
    z1 = z1 + b2_ref[0, 1]
    m = jnp.maximum(z0, z1)
    lse = m + jnp.log(jnp.exp(z0 - m) + jnp.exp(z1 - m))
    o0_ref[...] = z0 - lse
    o1_ref[...] = z1 - lse


def kernel(x, edge_index, W1, b1, W2, b2):
    n, d_in = x.shape
    e = edge_index.shape[1]
    d_hid = W1.shape[1]
    d_out = W2.shape[1]
    ep = _padded_edges(e)
    src = jnp.concatenate(
        [edge_index[0], jnp.zeros((ep - e,), jnp.int32)]).reshape(-1, _CH)
    dst = jnp.concatenate(
        [edge_index[1], jnp.full((ep - e,), n, jnp.int32)]).reshape(-1, _CH)

    ones16 = jnp.ones((_CH, _D), jnp.float32)
    zeros16 = jnp.zeros((n // 16, _D), jnp.float32)
    dn = n // 8

    deg_k = _make_deg_kernel(n, e)
    seg_k = _make_seg_kernel(n, e)

    degp = deg_k(dst, ones16, zeros16).reshape(2, dn, 128)
    d0 = degp[0]
    d1 = degp[1]

    x8 = x.reshape(dn, 8 * d_in)
    w1e = jnp.kron(jnp.eye(8, dtype=jnp.float32), W1)

    f32 = jnp.float32
    g1d, dinvd = pl.pallas_call(
        _tc1_body,
        out_shape=[jax.ShapeDtypeStruct((dn, 128), f32),
                   jax.ShapeDtypeStruct((dn, 128), f32)],
    )(x8, d0, d1, w1e)

    segp1 = seg_k(g1d.reshape(n, _D), src, dst,
                  zeros16).reshape(2, dn, 128)
    p0 = segp1[0]
    p1 = segp1[1]

    b1e = jnp.tile(b1, 128 // d_hid).reshape(1, 128)
    w2e = jnp.kron(jnp.eye(128 // _D, dtype=f32),
                   jnp.pad(W2, ((0, 0), (0, _D - d_out))))

    g2d = pl.pallas_call(
        _tc2_body,
        out_shape=jax.ShapeDtypeStruct((dn, 128), f32),
    )(p0, p1, g1d, dinvd, b1e, w2e)

    segp2 = seg_k(g2d.reshape(n, _D), src, dst,
                  zeros16).reshape(2, dn, 128)
    q0 = segp2[0]
    q1 = segp2[1]

    lane = jnp.arange(128)
    node = jnp.arange(8)
    e0 = (lane[:, None] == node[None, :] * _D).astype(f32)
    e1 = (lane[:, None] == node[None, :] * _D + 1).astype(f32)

    o0, o1 = pl.pallas_call(
        _tc3_body,
        out_shape=[jax.ShapeDtypeStruct((dn, 8), f32),
                   jax.ShapeDtypeStruct((dn, 8), f32)],
    )(q0, q1, g2d, dinvd, b2.reshape(1, d_out), e0, e1)

    return jnp.concatenate(
        [o0.reshape(n, 1), o1.reshape(n, 1)], axis=1)


# speedup vs baseline: 88.8653x; 1.5101x over previous
import functools

import jax
import jax.numpy as jnp
from jax import lax
from jax.experimental import pallas as pl
from jax.experimental.pallas import tpu as pltpu
from jax.experimental.pallas import tpu_sc as plsc

_CH = 128
_SUB = 8
_NW = 32
_BN = 2000
_D = 16


def _padded_edges(e):
    blk = _CH * _SUB * _NW
    return -(-e // blk) * blk


def _make_deg_kernel(n, e):
    k = _CH * _SUB
    ep = _padded_edges(e)
    itb = ep // (k * _NW)
    tpw = n // 16
    npad = n + 8
    mesh = plsc.VectorSubcoreMesh(core_axis_name="c", subcore_axis_name="s")

    @functools.partial(
        pl.kernel,
        out_type=jax.ShapeDtypeStruct((32, tpw, _D), jnp.float32),
        mesh=mesh,
        scratch_types=[
            pltpu.VMEM((2, _SUB, _CH), jnp.int32),
            pltpu.VMEM((_CH, _D), jnp.float32),
            pltpu.VMEM_SHARED((npad, _D), jnp.float32),
            pltpu.SemaphoreType.DMA,
            pltpu.SemaphoreType.DMA,
        ],
        compiler_params=pltpu.CompilerParams(use_tc_tiling_on_sc=False),
    )
    def deg_kernel(dst_hbm, ones_hbm, zeros_hbm, out_hbm,
                   didx, ones_v, acc, isem, ssem):
        cid = lax.axis_index("c")
        sid = lax.axis_index("s")
        wid = sid * 2 + cid
        r0 = sid * tpw
        pltpu.sync_copy(ones_hbm, ones_v)
        pltpu.sync_copy(zeros_hbm, acc.at[pl.ds(r0, tpw)])
        plsc.subcore_barrier()

        def fire_idx(i, slot):
            blk = wid + i * _NW
            pltpu.async_copy(
                dst_hbm.at[pl.ds(blk * _SUB, _SUB)], didx.at[slot], isem)

        def wait_idx(slot):
            pltpu.make_async_copy(
                dst_hbm.at[pl.ds(0, _SUB)], didx.at[slot], isem).wait()

        fire_idx(0, 0)

        @pl.loop(0, itb)
        def _(i):
            slot = lax.rem(i, 2)
            wait_idx(slot)

            @pl.when(i + 1 < itb)
            def _():
                fire_idx(i + 1, 1 - slot)

            descs = [pltpu.async_copy(ones_v, acc.at[didx.at[slot, j]],
                                      ssem, add=True) for j in range(_SUB)]
            for dsc in descs:
                dsc.wait()

        plsc.subcore_barrier()
        pltpu.sync_copy(acc.at[pl.ds(r0, tpw)], out_hbm.at[cid * 16 + sid])

    return deg_kernel


def _make_seg_kernel(n, e):
    k = _CH * _SUB
    ep = _padded_edges(e)
    itb = ep // (k * _NW)
    tpw = n // 16
    npad = n + 8
    mesh = plsc.VectorSubcoreMesh(core_axis_name="c", subcore_axis_name="s")

    @functools.partial(
        pl.kernel,
        out_type=jax.ShapeDtypeStruct((32, tpw, _D), jnp.float32),
        mesh=mesh,
        scratch_types=[
            pltpu.VMEM((2, _SUB, _CH), jnp.int32),
            pltpu.VMEM((2, _SUB, _CH), jnp.int32),
            pltpu.VMEM((k, _D), jnp.float32),
            pltpu.VMEM_SHARED((npad, _D), jnp.float32),
            pltpu.SemaphoreType.DMA,
            pltpu.SemaphoreType.DMA,
            pltpu.SemaphoreType.DMA,
        ],
        compiler_params=pltpu.CompilerParams(use_tc_tiling_on_sc=False),
    )
    def seg_kernel(g_hbm, src_hbm, dst_hbm, zeros_hbm, out_hbm,
                   sidx, didx, rows, acc, isem, gsem, ssem):
        cid = lax.axis_index("c")
        sid = lax.axis_index("s")
        wid = sid * 2 + cid
        r0 = sid * tpw
        pltpu.sync_copy(zeros_hbm, acc.at[pl.ds(r0, tpw)])
        plsc.subcore_barrier()

        def fire_idx(i, slot):
            blk = wid + i * _NW
            pltpu.async_copy(
                src_hbm.at[pl.ds(blk * _SUB, _SUB)], sidx.at[slot], isem)
            pltpu.async_copy(
                dst_hbm.at[pl.ds(blk * _SUB, _SUB)], didx.at[slot], isem)

        def wait_idx(slot):
            pltpu.make_async_copy(
                src_hbm.at[pl.ds(0, _SUB)], sidx.at[slot], isem).wait()
            pltpu.make_async_copy(
                dst_hbm.at[pl.ds(0, _SUB)], didx.at[slot], isem).wait()

        def fire_gathers(slot, js):
            return [pltpu.async_copy(g_hbm.at[sidx.at[slot, j]],
                                     rows.at[pl.ds(j * _CH, _CH)], gsem)
                    for j in js]

        def fire_scatters(slot, js):
            return [pltpu.async_copy(rows.at[pl.ds(j * _CH, _CH)],
                                     acc.at[didx.at[slot, j]], ssem, add=True)
                    for j in js]

        fire_idx(0, 0)

        @pl.loop(0, itb)
        def _(i):
            slot = lax.rem(i, 2)
            wait_idx(slot)

            @pl.when(i + 1 < itb)
            def _():
                fire_idx(i + 1, 1 - slot)

            half = _SUB // 2
            ga = fire_gathers(slot, range(half))
            for dsc in ga:
                dsc.wait()
            sa = fire_scatters(slot, range(half))
            gb = fire_gathers(slot, range(half, _SUB))
            for dsc in gb:
                dsc.wait()
            sb = fire_scatters(slot, range(half, _SUB))
            for dsc in sa + sb:
                dsc.wait()

        plsc.subcore_barrier()
        pltpu.sync_copy(acc.at[pl.ds(r0, tpw)], out_hbm.at[cid * 16 + sid])

    return seg_kernel


def _tc1_body(x8_ref, d_ref, w1e_ref, g1_ref, dinv_ref):
    dinv = lax.rsqrt(d_ref[0] + d_ref[1] + 1.0)
    h = jnp.dot(x8_ref[...], w1e_ref[...],
                preferred_element_type=jnp.float32)
    g1_ref[...] = dinv * h
    dinv_ref[...] = dinv


def _tc2_body(p_ref, g1_ref, dinv_ref, b1_ref, w2e_ref, g2_ref):
    dinv = dinv_ref[...]
    z = dinv * (p_ref[0] + p_ref[1] + g1_ref[...]) + b1_ref[...]
    h = jnp.maximum(z, 0.0)
    h2 = jnp.dot(h, w2e_ref[...], preferred_element_type=jnp.float32)
    g2_ref[...] = dinv * h2


def _tc3_body(q_ref, g2_ref, dinv_ref, b2_ref, e0_ref, e1_ref,
              o0_ref, o1_ref):
    s = dinv_ref[...] * (q_ref[0] + q_ref[1] + g2_ref[...])
    z0 = jnp.dot(s, e0_ref[...], preferred_element_type=jnp.float32)
    z1 = jnp.dot(s, e1_ref[...], preferred_element_type=jnp.float32)
    z0 = z0 + b2_ref[0, 0]
    z1 = z1 + b2_ref[0, 1]
    m = jnp.maximum(z0, z1)
    lse = m + jnp.log(jnp.exp(z0 - m) + jnp.exp(z1 - m))
    o0_ref[...] = z0 - lse
    o1_ref[...] = z1 - lse


def kernel(x, edge_index, W1, b1, W2, b2):
    n, d_in = x.shape
    e = edge_index.shape[1]
    d_hid = W1.shape[1]
    d_out = W2.shape[1]
    ep = _padded_edges(e)
    src = jnp.concatenate(
        [edge_index[0], jnp.zeros((ep - e,), jnp.int32)]).reshape(-1, _CH)
    dst = jnp.concatenate(
        [edge_index[1], jnp.full((ep - e,), n, jnp.int32)]).reshape(-1, _CH)

    ones16 = jnp.ones((_CH, _D), jnp.float32)
    zeros16 = jnp.zeros((n // 16, _D), jnp.float32)
    dn = n // 8

    deg_k = _make_deg_kernel(n, e)
    seg_k = _make_seg_kernel(n, e)

    degp = deg_k(dst, ones16, zeros16).ravel().reshape(2, dn, 128)

    x8 = x.reshape(dn, 8 * d_in)
    w1e = jnp.kron(jnp.eye(8, dtype=jnp.float32), W1)

    f32 = jnp.float32
    g1d, dinvd = pl.pallas_call(
        _tc1_body,
        out_shape=[jax.ShapeDtypeStruct((dn, 128), f32),
                   jax.ShapeDtypeStruct((dn, 128), f32)],
    )(x8, degp, w1e)

    segp1 = seg_k(g1d.reshape(n, _D), src, dst,
                  zeros16).ravel().reshape(2, dn, 128)

    b1e = jnp.tile(b1, 128 // d_hid).reshape(1, 128)
    w2e = jnp.kron(jnp.eye(128 // _D, dtype=f32),
                   jnp.pad(W2, ((0, 0), (0, _D - d_out))))

    g2d = pl.pallas_call(
        _tc2_body,
        out_shape=jax.ShapeDtypeStruct((dn, 128), f32),
    )(segp1, g1d, dinvd, b1e, w2e)

    segp2 = seg_k(g2d.reshape(n, _D), src, dst,
                  zeros16).ravel().reshape(2, dn, 128)

    lane = jnp.arange(128)
    node = jnp.arange(8)
    e0 = (lane[:, None] == node[None, :] * _D).astype(f32)
    e1 = (lane[:, None] == node[None, :] * _D + 1).astype(f32)

    o0, o1 = pl.pallas_call(
        _tc3_body,
        out_shape=[jax.ShapeDtypeStruct((dn, 8), f32),
                   jax.ShapeDtypeStruct((dn, 8), f32)],
    )(segp2, g2d, dinvd, b2.reshape(1, d_out), e0, e1)

    return jnp.concatenate(
        [o0.reshape(n, 1), o1.reshape(n, 1)], axis=1)


# speedup vs baseline: 106.0948x; 1.1939x over previous
import functools

import jax
import jax.numpy as jnp
from jax import lax
from jax.experimental import pallas as pl
from jax.experimental.pallas import tpu as pltpu
from jax.experimental.pallas import tpu_sc as plsc

_CH = 128
_SUB = 8
_NW = 32
_BN = 2000
_D = 16


def _padded_edges(e):
    blk = _CH * _SUB * _NW
    return -(-e // blk) * blk


def _make_deg_kernel(n, e):
    k = _CH * _SUB
    ep = _padded_edges(e)
    itb = ep // (k * _NW)
    tpw = n // 16
    npad = n + 8
    mesh = plsc.VectorSubcoreMesh(core_axis_name="c", subcore_axis_name="s")

    @functools.partial(
        pl.kernel,
        out_type=jax.ShapeDtypeStruct((32, tpw, _D), jnp.float32),
        mesh=mesh,
        scratch_types=[
            pltpu.VMEM((2, _SUB, _CH), jnp.int32),
            pltpu.VMEM((_CH, _D), jnp.float32),
            pltpu.VMEM_SHARED((npad, _D), jnp.float32),
            pltpu.SemaphoreType.DMA,
            pltpu.SemaphoreType.DMA,
        ],
        compiler_params=pltpu.CompilerParams(use_tc_tiling_on_sc=False),
    )
    def deg_kernel(dst_hbm, ones_hbm, zeros_hbm, out_hbm,
                   didx, ones_v, acc, isem, ssem):
        cid = lax.axis_index("c")
        sid = lax.axis_index("s")
        wid = sid * 2 + cid
        r0 = sid * tpw
        pltpu.sync_copy(ones_hbm, ones_v)
        pltpu.sync_copy(zeros_hbm, acc.at[pl.ds(r0, tpw)])
        plsc.subcore_barrier()

        def fire_idx(i, slot):
            blk = wid + i * _NW
            pltpu.async_copy(
                dst_hbm.at[pl.ds(blk * _SUB, _SUB)], didx.at[slot], isem)

        def wait_idx(slot):
            pltpu.make_async_copy(
                dst_hbm.at[pl.ds(0, _SUB)], didx.at[slot], isem).wait()

        fire_idx(0, 0)

        @pl.loop(0, itb)
        def _(i):
            slot = lax.rem(i, 2)
            wait_idx(slot)

            @pl.when(i + 1 < itb)
            def _():
                fire_idx(i + 1, 1 - slot)

            descs = [pltpu.async_copy(ones_v, acc.at[didx.at[slot, j]],
                                      ssem, add=True) for j in range(_SUB)]
            for dsc in descs:
                dsc.wait()

        plsc.subcore_barrier()
        pltpu.sync_copy(acc.at[pl.ds(r0, tpw)], out_hbm.at[cid * 16 + sid])

    return deg_kernel


def _make_seg_kernel(n, e):
    k = _CH * _SUB
    ep = _padded_edges(e)
    itb = ep // (k * _NW)
    tpw = n // 16
    npad = n + 8
    mesh = plsc.VectorSubcoreMesh(core_axis_name="c", subcore_axis_name="s")

    @functools.partial(
        pl.kernel,
        out_type=jax.ShapeDtypeStruct((32, tpw, _D), jnp.float32),
        mesh=mesh,
        scratch_types=[
            pltpu.VMEM((2, _SUB, _CH), jnp.int32),
            pltpu.VMEM((2, _SUB, _CH), jnp.int32),
            pltpu.VMEM((k, _D), jnp.float32),
            pltpu.VMEM_SHARED((npad, _D), jnp.float32),
            pltpu.SemaphoreType.DMA,
            pltpu.SemaphoreType.DMA,
            pltpu.SemaphoreType.DMA,
        ],
        compiler_params=pltpu.CompilerParams(use_tc_tiling_on_sc=False),
    )
    def seg_kernel(g_hbm, src_hbm, dst_hbm, zeros_hbm, out_hbm,
                   sidx, didx, rows, acc, isem, gsem, ssem):
        cid = lax.axis_index("c")
        sid = lax.axis_index("s")
        wid = sid * 2 + cid
        r0 = sid * tpw
        pltpu.sync_copy(zeros_hbm, acc.at[pl.ds(r0, tpw)])
        plsc.subcore_barrier()

        def fire_idx(i, slot):
            blk = wid + i * _NW
            pltpu.async_copy(
                src_hbm.at[pl.ds(blk * _SUB, _SUB)], sidx.at[slot], isem)
            pltpu.async_copy(
                dst_hbm.at[pl.ds(blk * _SUB, _SUB)], didx.at[slot], isem)

        def wait_idx(slot):
            pltpu.make_async_copy(
                src_hbm.at[pl.ds(0, _SUB)], sidx.at[slot], isem).wait()
            pltpu.make_async_copy(
                dst_hbm.at[pl.ds(0, _SUB)], didx.at[slot], isem).wait()

        def fire_gathers(slot, js):
            return [pltpu.async_copy(g_hbm.at[sidx.at[slot, j]],
                                     rows.at[pl.ds(j * _CH, _CH)], gsem)
                    for j in js]

        def fire_scatters(slot, js):
            return [pltpu.async_copy(rows.at[pl.ds(j * _CH, _CH)],
                                     acc.at[didx.at[slot, j]], ssem, add=True)
                    for j in js]

        fire_idx(0, 0)

        @pl.loop(0, itb)
        def _(i):
            slot = lax.rem(i, 2)
            wait_idx(slot)

            @pl.when(i + 1 < itb)
            def _():
                fire_idx(i + 1, 1 - slot)

            ga = fire_gathers(slot, range(_SUB))
            sa = []
            for j in range(_SUB):
                ga[j].wait()
                sa += fire_scatters(slot, [j])
            for dsc in sa:
                dsc.wait()

        plsc.subcore_barrier()
        pltpu.sync_copy(acc.at[pl.ds(r0, tpw)], out_hbm.at[cid * 16 + sid])

    return seg_kernel


def _tc1_body(x8_ref, d_ref, w1e_ref, g1_ref, dinv_ref):
    dinv = lax.rsqrt(d_ref[0] + d_ref[1] + 1.0)
    h = jnp.dot(x8_ref[...], w1e_ref[...],
                preferred_element_type=jnp.float32)
    g1_ref[...] = dinv * h
    dinv_ref[...] = dinv


def _tc2_body(p_ref, g1_ref, dinv_ref, b1_ref, w2e_ref, g2_ref):
    dinv = dinv_ref[...]
    z = dinv * (p_ref[0] + p_ref[1] + g1_ref[...]) + b1_ref[...]
    h = jnp.maximum(z, 0.0)
    h2 = jnp.dot(h, w2e_ref[...], preferred_element_type=jnp.float32)
    g2_ref[...] = dinv * h2


def _tc3_body(q_ref, g2_ref, dinv_ref, b2_ref, e0_ref, e1_ref,
              o0_ref, o1_ref):
    s = dinv_ref[...] * (q_ref[0] + q_ref[1] + g2_ref[...])
    z0 = jnp.dot(s, e0_ref[...], preferred_element_type=jnp.float32)
    z1 = jnp.dot(s, e1_ref[...], preferred_element_type=jnp.float32)
    z0 = z0 + b2_ref[0, 0]
    z1 = z1 + b2_ref[0, 1]
    m = jnp.maximum(z0, z1)
    lse = m + jnp.log(jnp.exp(z0 - m) + jnp.exp(z1 - m))
    o0_ref[...] = z0 - lse
    o1_ref[...] = z1 - lse


def kernel(x, edge_index, W1, b1, W2, b2):
    n, d_in = x.shape
    e = edge_index.shape[1]
    d_hid = W1.shape[1]
    d_out = W2.shape[1]
    ep = _padded_edges(e)
    src = jnp.concatenate(
        [edge_index[0], jnp.zeros((ep - e,), jnp.int32)]).reshape(-1, _CH)
    dst = jnp.concatenate(
        [edge_index[1], jnp.full((ep - e,), n, jnp.int32)]).reshape(-1, _CH)

    ones16 = jnp.ones((_CH, _D), jnp.float32)
    zeros16 = jnp.zeros((n // 16, _D), jnp.float32)
    dn = n // 8

    deg_k = _make_deg_kernel(n, e)
    seg_k = _make_seg_kernel(n, e)

    degp = deg_k(dst, ones16, zeros16).ravel().reshape(2, dn, 128)

    x8 = x.reshape(dn, 8 * d_in)
    w1e = jnp.kron(jnp.eye(8, dtype=jnp.float32), W1)

    f32 = jnp.float32
    g1d, dinvd = pl.pallas_call(
        _tc1_body,
        out_shape=[jax.ShapeDtypeStruct((dn, 128), f32),
                   jax.ShapeDtypeStruct((dn, 128), f32)],
    )(x8, degp, w1e)

    segp1 = seg_k(g1d.reshape(n, _D), src, dst,
                  zeros16).ravel().reshape(2, dn, 128)

    b1e = jnp.tile(b1, 128 // d_hid).reshape(1, 128)
    w2e = jnp.kron(jnp.eye(128 // _D, dtype=f32),
                   jnp.pad(W2, ((0, 0), (0, _D - d_out))))

    g2d = pl.pallas_call(
        _tc2_body,
        out_shape=jax.ShapeDtypeStruct((dn, 128), f32),
    )(segp1, g1d, dinvd, b1e, w2e)

    segp2 = seg_k(g2d.reshape(n, _D), src, dst,
                  zeros16).ravel().reshape(2, dn, 128)

    lane = jnp.arange(128)
    node = jnp.arange(8)
    e0 = (lane[:, None] == node[None, :] * _D).astype(f32)
    e1 = (lane[:, None] == node[None, :] * _D + 1).astype(f32)

    o0, o1 = pl.pallas_call(
        _tc3_body,
        out_shape=[jax.ShapeDtypeStruct((dn, 8), f32),
                   jax.ShapeDtypeStruct((dn, 8), f32)],
    )(segp2, g2d, dinvd, b2.reshape(1, d_out), e0, e1)

    return jnp.concatenate(
        [o0.reshape(n, 1), o1.reshape(n, 1)], axis=1)
